# baseline (speedup 1.0000x reference)
import functools
import math

import jax
import jax.numpy as jnp
from jax import lax
from jax.experimental import pallas as pl
from jax.experimental.pallas import tpu as pltpu
from jax.experimental.pallas import tpu_sc as plsc

_SENT = 4096
_SLEN = 50
_SPAD = 56
_EMB = 512
_SCALE = math.sqrt(_EMB)
_LANES = 16

_NC = 2
_NS = 16
_NW = _NC * _NS

_SPW = _SENT // _NW
_IPW = _SPW * _SPAD
_NBUF = 4
_OUTER = _SPW // _NBUF


def _make_sc_kernel():
  mesh = plsc.VectorSubcoreMesh(core_axis_name="c", subcore_axis_name="s")

  @functools.partial(
      pl.kernel,
      out_type=jax.ShapeDtypeStruct((_SENT * _SPAD, _EMB), jnp.float32),
      mesh=mesh,
      scratch_types=(
          [pltpu.VMEM((_IPW,), jnp.int32),
           pltpu.VMEM((_NBUF, _SPAD, _EMB), jnp.float32)]
          + [pltpu.SemaphoreType.DMA] * (2 * _NBUF)
      ),
  )
  def sc_embed(idx_hbm, table_hbm, out_hbm, idx_v, buf, *sems):
    g_sems = sems[:_NBUF]
    o_sems = sems[_NBUF:]
    wid = lax.axis_index("s") * _NC + lax.axis_index("c")
    sent_base = wid * _SPW
    pltpu.sync_copy(idx_hbm.at[pl.ds(wid * _IPW, _IPW)], idx_v)

    def gather_copy(i, b):
      return pltpu.make_async_copy(
          table_hbm.at[idx_v.at[pl.ds(i * _SPAD, _SPAD)]],
          buf.at[b], g_sems[b])

    def out_copy(i, b):
      return pltpu.make_async_copy(
          buf.at[b], out_hbm.at[pl.ds((sent_base + i) * _SPAD, _SPAD)],
          o_sems[b])

    gather_copy(0, 0).start()
    gather_copy(1, 1).start()

    def outer(j, carry):
      for b in range(_NBUF):
        i = j * _NBUF + b
        gather_copy(i, b).wait()

        def scale_row(r, c2, _b=b):
          for c in range(_EMB // _LANES):
            buf[_b, r, pl.ds(c * _LANES, _LANES)] = (
                buf[_b, r, pl.ds(c * _LANES, _LANES)] * _SCALE)
          return c2
        lax.fori_loop(0, _SLEN, scale_row, 0)

        out_copy(i, b).start()

        bn = (b + 2) % _NBUF
        if b < 2:
          @pl.when(j >= 1)
          def _(i=i, bn=bn):
            out_copy(i - 2, bn).wait()
          gather_copy(i + 2, bn).start()
        else:
          @pl.when(j < _OUTER - 1)
          def _(i=i, bn=bn):
            out_copy(i - 2, bn).wait()
            gather_copy(i + 2, bn).start()
      return carry

    lax.fori_loop(0, _OUTER, outer, 0)

    for b in range(_NBUF):
      out_copy(_SPW - _NBUF + b, b).wait()

  return sc_embed


_SC_EMBED = _make_sc_kernel()


def kernel(x, table):
  xp = jnp.pad(x, ((0, 0), (0, _SPAD - _SLEN)))
  out = _SC_EMBED(xp.reshape(-1), table)
  return out.reshape(_SENT, _SPAD, _EMB)[:, :_SLEN, :]


# speedup vs baseline: 1.5235x; 1.5235x over previous
import functools
import math

import jax
import jax.numpy as jnp
from jax import lax
from jax.experimental import pallas as pl
from jax.experimental.pallas import tpu as pltpu
from jax.experimental.pallas import tpu_sc as plsc

_EMB = 512
_SCALE = math.sqrt(_EMB)
_LANES = 16

_NC = 2
_NS = 16
_NW = _NC * _NS

_B = 4096 * 50
_BPW = _B // _NW
_C = 40
_NCHUNK = _BPW // _C
_NBUF = 4
_OUTER = _NCHUNK // _NBUF


def _make_sc_kernel():
  mesh = plsc.VectorSubcoreMesh(core_axis_name="c", subcore_axis_name="s")

  @functools.partial(
      pl.kernel,
      out_type=jax.ShapeDtypeStruct((_B, _EMB), jnp.float32),
      mesh=mesh,
      scratch_types=(
          [pltpu.VMEM((_BPW,), jnp.int32),
           pltpu.VMEM((_NBUF, _C, _EMB), jnp.float32)]
          + [pltpu.SemaphoreType.DMA] * (2 * _NBUF)
      ),
  )
  def sc_embed(idx_hbm, table_hbm, out_hbm, idx_v, buf, *sems):
    g_sems = sems[:_NBUF]
    o_sems = sems[_NBUF:]
    wid = lax.axis_index("s") * _NC + lax.axis_index("c")
    base = wid * _BPW
    pltpu.sync_copy(idx_hbm.at[pl.ds(base, _BPW)], idx_v)

    def gather_copy(i, b):
      return pltpu.make_async_copy(
          table_hbm.at[idx_v.at[pl.ds(i * _C, _C)]], buf.at[b], g_sems[b])

    def out_copy(i, b):
      return pltpu.make_async_copy(
          buf.at[b], out_hbm.at[pl.ds(base + i * _C, _C)], o_sems[b])

    gather_copy(0, 0).start()
    gather_copy(1, 1).start()

    def outer(j, carry):
      for b in range(_NBUF):
        i = j * _NBUF + b
        gather_copy(i, b).wait()
        out_copy(i, b).start()

        bn = (b + 2) % _NBUF
        if b < 2:
          @pl.when(j >= 1)
          def _(i=i, b=b, bn=bn):
            out_copy(i - 2, bn).wait()
          gather_copy(i + 2, bn).start()
        else:
          @pl.when(j < _OUTER - 1)
          def _(i=i, b=b, bn=bn):
            out_copy(i - 2, bn).wait()
            gather_copy(i + 2, bn).start()
      return carry

    lax.fori_loop(0, _OUTER, outer, 0)

    for b in range(_NBUF):
      out_copy(_NCHUNK - _NBUF + b, b).wait()

  return sc_embed


_SC_EMBED = _make_sc_kernel()

_SENT = 4096
_SLEN = 50
_TC_GRP = 16
_TC_BLK = _TC_GRP * _SLEN * _EMB // (8 * 128)


def _tc_scale_relayout(t):
  def body(i_ref, o_ref):
    o_ref[...] = i_ref[...].reshape(o_ref.shape) * _SCALE

  return pl.pallas_call(
      body,
      grid=(_SENT // _TC_GRP,),
      in_specs=[pl.BlockSpec((_TC_BLK, 8, 128), lambda i: (i, 0, 0))],
      out_specs=pl.BlockSpec((_TC_GRP, _SLEN, _EMB), lambda i: (i, 0, 0)),
      out_shape=jax.ShapeDtypeStruct((_SENT, _SLEN, _EMB), jnp.float32),
  )(t)


def kernel(x, table):
  idx_flat = x.reshape(-1)
  out = _SC_EMBED(idx_flat, table)
  t = out.reshape(_B * _EMB // (8 * 128), 8, 128)
  return _tc_scale_relayout(t)


# speedup vs baseline: 1.6541x; 1.0858x over previous
import functools
import math

import jax
import jax.numpy as jnp
from jax import lax
from jax.experimental import pallas as pl
from jax.experimental.pallas import tpu as pltpu
from jax.experimental.pallas import tpu_sc as plsc

_EMB = 512
_SEG = 128
_SPR = _EMB // _SEG
_SCALE = math.sqrt(_EMB)
_LANES = 16

_NC = 2
_NS = 16
_NW = _NC * _NS

_B = 4096 * 50
_NSEG = _B * _SPR
_SEGW = _NSEG // _NW
_C = 160
_NCHUNK = _SEGW // _C
_NBUF = 4
_OUTER = _NCHUNK // _NBUF


def _make_sc_kernel():
  mesh = plsc.VectorSubcoreMesh(core_axis_name="c", subcore_axis_name="s")

  @functools.partial(
      pl.kernel,
      out_type=jax.ShapeDtypeStruct((_NSEG, _SEG), jnp.float32),
      mesh=mesh,
      scratch_types=(
          [pltpu.VMEM((_SEGW,), jnp.int32),
           pltpu.VMEM((_NBUF, _C, _SEG), jnp.float32)]
          + [pltpu.SemaphoreType.DMA] * (2 * _NBUF)
      ),
  )
  def sc_embed(idx_hbm, tseg_hbm, out_hbm, idx_v, buf, *sems):
    g_sems = sems[:_NBUF]
    o_sems = sems[_NBUF:]
    wid = lax.axis_index("s") * _NC + lax.axis_index("c")
    base = wid * _SEGW
    pltpu.sync_copy(idx_hbm.at[pl.ds(base, _SEGW)], idx_v)

    def gather_copy(i, b):
      return pltpu.make_async_copy(
          tseg_hbm.at[idx_v.at[pl.ds(i * _C, _C)]], buf.at[b], g_sems[b])

    def out_copy(i, b):
      return pltpu.make_async_copy(
          buf.at[b], out_hbm.at[pl.ds(base + i * _C, _C)], o_sems[b])

    gather_copy(0, 0).start()
    gather_copy(1, 1).start()

    def outer(j, carry):
      for b in range(_NBUF):
        i = j * _NBUF + b
        gather_copy(i, b).wait()

        def scale_grp(r, c2, _b=b):
          for k in range(64):
            rr, c = divmod(k, 8)
            buf[_b, r * 8 + rr, pl.ds(c * _LANES, _LANES)] = (
                buf[_b, r * 8 + rr, pl.ds(c * _LANES, _LANES)] * _SCALE)
          return c2
        lax.fori_loop(0, _C // 8, scale_grp, 0)

        out_copy(i, b).start()

        bn = (b + 2) % _NBUF
        if b < 2:
          @pl.when(j >= 1)
          def _(i=i, b=b, bn=bn):
            out_copy(i - 2, bn).wait()
          gather_copy(i + 2, bn).start()
        else:
          @pl.when(j < _OUTER - 1)
          def _(i=i, b=b, bn=bn):
            out_copy(i - 2, bn).wait()
            gather_copy(i + 2, bn).start()
      return carry

    lax.fori_loop(0, _OUTER, outer, 0)

    for b in range(_NBUF):
      out_copy(_NCHUNK - _NBUF + b, b).wait()

  return sc_embed


_SC_EMBED = _make_sc_kernel()


def kernel(x, table):
  idx = x.reshape(-1)
  idx_seg = (idx[:, None] * _SPR + jnp.arange(_SPR, dtype=idx.dtype))
  tseg = table.reshape(-1, _SEG)
  out = _SC_EMBED(idx_seg.reshape(-1), tseg)
  return out.reshape(x.shape + (table.shape[1],))


# speedup vs baseline: 3.2672x; 1.9752x over previous
import functools
import math

import jax
import jax.numpy as jnp
from jax import lax
from jax.experimental import pallas as pl
from jax.experimental.pallas import tpu as pltpu
from jax.experimental.pallas import tpu_sc as plsc

_SENT = 4096
_SLEN = 50
_SPAD = 56
_EMB = 512
_SCALE = math.sqrt(_EMB)
_LANES = 16

_NC = 2
_NS = 16
_NW = _NC * _NS

_SPW = _SENT // _NW
_IPW = _SPW * _SPAD
_NBUF = 4
_OUTER = _SPW // _NBUF


def _make_sc_kernel():
  mesh = plsc.VectorSubcoreMesh(core_axis_name="c", subcore_axis_name="s")

  @functools.partial(
      pl.kernel,
      out_type=jax.ShapeDtypeStruct((_SENT * _SPAD, _EMB), jnp.float32),
      mesh=mesh,
      scratch_types=(
          [pltpu.VMEM((_IPW,), jnp.int32),
           pltpu.VMEM((_NBUF, _SPAD, _EMB), jnp.float32)]
          + [pltpu.SemaphoreType.DMA] * (2 * _NBUF)
      ),
  )
  def sc_embed(idx_hbm, table_hbm, out_hbm, idx_v, buf, *sems):
    g_sems = sems[:_NBUF]
    o_sems = sems[_NBUF:]
    wid = lax.axis_index("s") * _NC + lax.axis_index("c")
    sent_base = wid * _SPW
    pltpu.sync_copy(idx_hbm.at[pl.ds(wid * _IPW, _IPW)], idx_v)

    def gather_copy(i, b):
      return pltpu.make_async_copy(
          table_hbm.at[idx_v.at[pl.ds(i * _SPAD, _SPAD)]],
          buf.at[b], g_sems[b])

    def out_copy(i, b):
      return pltpu.make_async_copy(
          buf.at[b], out_hbm.at[pl.ds((sent_base + i) * _SPAD, _SPAD)],
          o_sems[b])

    gather_copy(0, 0).start()
    gather_copy(1, 1).start()

    def outer(j, carry):
      for b in range(_NBUF):
        i = j * _NBUF + b
        gather_copy(i, b).wait()

        def scale_row(r, c2, _b=b):
          for c in range(_EMB // _LANES):
            buf[_b, r, pl.ds(c * _LANES, _LANES)] = (
                buf[_b, r, pl.ds(c * _LANES, _LANES)] * _SCALE)
          return c2
        lax.fori_loop(0, _SLEN, scale_row, 0)

        out_copy(i, b).start()

        bn = (b + 2) % _NBUF
        if b < 2:
          @pl.when(j >= 1)
          def _(i=i, bn=bn):
            out_copy(i - 2, bn).wait()
          gather_copy(i + 2, bn).start()
        else:
          @pl.when(j < _OUTER - 1)
          def _(i=i, bn=bn):
            out_copy(i - 2, bn).wait()
            gather_copy(i + 2, bn).start()
      return carry

    lax.fori_loop(0, _OUTER, outer, 0)

    for b in range(_NBUF):
      out_copy(_SPW - _NBUF + b, b).wait()

  return sc_embed


_SC_EMBED = _make_sc_kernel()


def kernel(x, table):
  xp = jnp.concatenate([x, x[:, : _SPAD - _SLEN]], axis=1)
  out = _SC_EMBED(xp.reshape(-1), table)
  return out.reshape(_SENT, _SPAD, _EMB)[:, :_SLEN, :]
